# baseline (speedup 1.0000x reference)
import functools

import jax
import jax.numpy as jnp
from jax import lax
from jax.experimental import pallas as pl
from jax.experimental.pallas import tpu as pltpu
from jax.experimental.pallas import tpu_sc as plsc

NC = 2
NS = 16
L = 16
NW = NC * NS

G = 128
EMBED = 64


def _sc_gather_dot(uidx2d, midx2d, ue, me, ub_flat, mb_flat, batch):
  chunk = batch // NW
  nsplit = chunk // G
  rows_per_w = chunk // G
  mesh = plsc.VectorSubcoreMesh(core_axis_name="c", subcore_axis_name="s")

  @functools.partial(
      pl.kernel,
      out_type=(
          jax.ShapeDtypeStruct((NW, L), jnp.float32),
          jax.ShapeDtypeStruct((batch // G, G), jnp.float32)
      ),
      mesh=mesh,
      compiler_params=pltpu.CompilerParams(use_tc_tiling_on_sc=False),
      scratch_types=[
          pltpu.VMEM((nsplit, G), jnp.int32),
          pltpu.VMEM((nsplit, G), jnp.int32),
          pltpu.VMEM((chunk, EMBED), jnp.float32),
          pltpu.VMEM((chunk, EMBED), jnp.float32),
          pltpu.VMEM((nsplit, G), jnp.float32),
          pltpu.VMEM((nsplit, G), jnp.float32),
          pltpu.VMEM((L,), jnp.float32),
          pltpu.SemaphoreType.DMA,
          pltpu.SemaphoreType.DMA,
          pltpu.SemaphoreType.DMA,
          pltpu.SemaphoreType.DMA,
      ],
  )
  def k1(uidx_hbm, midx_hbm, ue_hbm, me_hbm, ub_hbm, mb_hbm,
         part_hbm, bsum_hbm,
         uidx_v, midx_v, urows_v, mrows_v, ubv, mbv, accv,
         sem_u, sem_m, sem_ub, sem_mb):
    wid = lax.axis_index("s") * NC + lax.axis_index("c")
    row0 = wid * rows_per_w
    pltpu.sync_copy(uidx_hbm.at[pl.ds(row0, nsplit)], uidx_v)
    pltpu.sync_copy(midx_hbm.at[pl.ds(row0, nsplit)], midx_v)

    copies = []
    for j in range(nsplit):
      copies.append(pltpu.async_copy(
          ue_hbm.at[uidx_v.at[j]], urows_v.at[pl.ds(j * G, G)], sem_u))
      copies.append(pltpu.async_copy(
          me_hbm.at[midx_v.at[j]], mrows_v.at[pl.ds(j * G, G)], sem_m))
      copies.append(pltpu.async_copy(
          ub_hbm.at[uidx_v.at[j]], ubv.at[j], sem_ub))
      copies.append(pltpu.async_copy(
          mb_hbm.at[midx_v.at[j]], mbv.at[j], sem_mb))
    for c in copies:
      c.wait()

    def bias_body(i, _):
      r = i // (G // L)
      o = (i % (G // L)) * L
      ubv[r, pl.ds(o, L)] = ubv[r, pl.ds(o, L)] + mbv[r, pl.ds(o, L)]
      return 0
    lax.fori_loop(0, nsplit * (G // L), bias_body, 0, unroll=4)
    pltpu.sync_copy(ubv, bsum_hbm.at[pl.ds(row0, nsplit)])

    def dot_body(i, accs):
      a0, a1, a2, a3 = accs
      a0 = a0 + urows_v[i, pl.ds(0, L)] * mrows_v[i, pl.ds(0, L)]
      a1 = a1 + urows_v[i, pl.ds(L, L)] * mrows_v[i, pl.ds(L, L)]
      a2 = a2 + urows_v[i, pl.ds(2 * L, L)] * mrows_v[i, pl.ds(2 * L, L)]
      a3 = a3 + urows_v[i, pl.ds(3 * L, L)] * mrows_v[i, pl.ds(3 * L, L)]
      return (a0, a1, a2, a3)
    zero = jnp.zeros((L,), jnp.float32)
    a0, a1, a2, a3 = lax.fori_loop(0, chunk, dot_body,
                                   (zero, zero, zero, zero), unroll=2)
    accv[...] = (a0 + a1) + (a2 + a3)
    pltpu.sync_copy(accv, part_hbm.at[wid])

  return k1(uidx2d, midx2d, ue, me, ub_flat, mb_flat)


def _tc_finish(part_ref, bsum_ref, out_ref):
  s = jnp.sum(part_ref[...])
  out_ref[...] = jax.nn.sigmoid(bsum_ref[...] + s)


def kernel(inputs, user_embedding, user_bias, movie_embedding, movie_bias):
  batch = inputs.shape[0]
  uidx2d = inputs[:, 0].reshape(batch // G, G)
  midx2d = inputs[:, 1].reshape(batch // G, G)
  ub_flat = user_bias.reshape(-1)
  mb_flat = movie_bias.reshape(-1)

  partials, bsum = _sc_gather_dot(
      uidx2d, midx2d, user_embedding, movie_embedding, ub_flat, mb_flat,
      batch)

  out = pl.pallas_call(
      _tc_finish,
      out_shape=jax.ShapeDtypeStruct(bsum.shape, jnp.float32),
  )(partials, bsum)
  return out.reshape(batch, 1)
